# baseline (speedup 1.0000x reference)
import functools

import jax
import jax.numpy as jnp
from jax import lax
from jax.experimental import pallas as pl
from jax.experimental.pallas import tpu as pltpu
from jax.experimental.pallas import tpu_sc as plsc

DIM = 768
HID = 1536
E = 64
N = 32768

NC = 2
NS = 16
NW = NC * NS
CHUNK = N // NW

BM = 256
MAX_TILES = N // BM + E
MPAD = MAX_TILES * BM
SLOTS_H = MPAD // NW
GR = 64
S3_PAIRS = SLOTS_H // (2 * GR)

_mesh = plsc.VectorSubcoreMesh(core_axis_name="c", subcore_axis_name="s")


def _wid():
    return lax.axis_index("s") * NC + lax.axis_index("c")


SC_BASE = 1


@functools.partial(
    pl.kernel,
    mesh=_mesh,
    out_type=jax.ShapeDtypeStruct((NW, E), jnp.int32),
    scratch_types=[
        pltpu.VMEM((CHUNK,), jnp.int32),
        pltpu.VMEM((E,), jnp.int32),
    ],
    compiler_params=pltpu.CompilerParams(needs_layout_passes=False),
)
def _s1_count(e_hbm, counts_hbm, e_v, cnt_v):
    w = _wid()
    pltpu.sync_copy(e_hbm.at[pl.ds(w * CHUNK, CHUNK)], e_v)
    zeros = jnp.zeros((16,), jnp.int32)
    for j in range(E // 16):
        cnt_v[pl.ds(j * 16, 16)] = zeros

    def body(k, carry):
        e16 = e_v[pl.ds(k * 16, 16)]
        occ, last = plsc.scan_count(e16)
        old = plsc.load_gather(cnt_v, [e16])
        plsc.store_scatter(cnt_v, [e16], old + occ + (1 - SC_BASE), mask=last)
        return carry

    lax.fori_loop(0, CHUNK // 16, body, 0)
    pltpu.sync_copy(cnt_v, counts_hbm.at[w])


@functools.partial(
    pl.kernel,
    mesh=_mesh,
    out_type=(
        jax.ShapeDtypeStruct((MPAD,), jnp.int32),
        jax.ShapeDtypeStruct((MPAD,), jnp.float32),
        jax.ShapeDtypeStruct((NW, 8, 128), jnp.int32),
    ),
    scratch_types=[
        pltpu.VMEM((CHUNK,), jnp.int32),
        pltpu.VMEM((E,), jnp.int32),
        pltpu.VMEM((8, 128), jnp.int32),
        pltpu.VMEM((8, 128), jnp.int32),
        pltpu.VMEM((8, 128), jnp.int32),
        pltpu.VMEM((8, 128), jnp.float32),
        pltpu.SemaphoreType.DMA,
    ],
    compiler_params=pltpu.CompilerParams(needs_layout_passes=False),
)
def _s2_place(e_hbm, rwflat_hbm, offs_hbm, perm_hbm, gatep_hbm, pos_hbm,
              e_v, cur_v, pos_v, tok_v, gidx_v, gate_v, sem):
    w = _wid()
    base = w * CHUNK
    pltpu.sync_copy(e_hbm.at[pl.ds(base, CHUNK)], e_v)
    pltpu.sync_copy(offs_hbm.at[w], cur_v)

    iota16 = lax.iota(jnp.int32, 16)
    copies = []
    for k in range(CHUNK // 16):
        e16 = e_v[pl.ds(k * 16, 16)]
        occ, last = plsc.scan_count(e16)
        old = plsc.load_gather(cur_v, [e16])
        pos16 = old + occ - SC_BASE
        plsc.store_scatter(cur_v, [e16], old + occ + (1 - SC_BASE), mask=last)
        loc16 = iota16 + (k * 16)
        tok16 = loc16 + base
        r, c = k // 8, (k % 8) * 16
        pos_v[r, pl.ds(c, 16)] = pos16
        tok_v[r, pl.ds(c, 16)] = tok16
        gidx_v[r, pl.ds(c, 16)] = tok16 * E + e16
        if k % 8 == 7:
            copies.append(pltpu.async_copy(
                rwflat_hbm.at[gidx_v.at[r]], gate_v.at[r], sem))
            copies.append(pltpu.async_copy(
                tok_v.at[r], perm_hbm.at[pos_v.at[r]], sem))
    for cp in copies:
        cp.wait()

    copies = [
        pltpu.async_copy(gate_v.at[j], gatep_hbm.at[pos_v.at[j]], sem)
        for j in range(8)
    ]
    for cp in copies:
        cp.wait()
    pltpu.sync_copy(pos_v, pos_hbm.at[w])


SROWS = SLOTS_H // 128


@functools.partial(
    pl.kernel,
    mesh=_mesh,
    out_type=jax.ShapeDtypeStruct((MPAD, DIM), jnp.float32),
    scratch_types=[
        pltpu.VMEM((SROWS, 128), jnp.int32),
        pltpu.VMEM((128, DIM), jnp.float32),
        pltpu.SemaphoreType.DMA,
    ],
)
def _s3_gather_x(perm_hbm, x_hbm, xp_hbm, perm_v, rows_v, sem):
    w = _wid()
    sbase = w * SLOTS_H
    pltpu.sync_copy(perm_hbm.at[w], perm_v)
    for r in range(SROWS):
        for k in range(8):
            v = perm_v[r, pl.ds(k * 16, 16)]
            perm_v[r, pl.ds(k * 16, 16)] = jnp.clip(v, 0, N - 1)

    def body(j, carry):
        pltpu.async_copy(x_hbm.at[perm_v.at[j]], rows_v, sem).wait()
        pltpu.sync_copy(rows_v, xp_hbm.at[pl.ds(sbase + j * 128, 128)])
        return carry

    lax.fori_loop(0, SROWS, body, 0)


@functools.partial(
    pl.kernel,
    mesh=_mesh,
    out_type=jax.ShapeDtypeStruct((N, DIM), jnp.float32),
    scratch_types=[
        pltpu.VMEM((8, 128), jnp.int32),
        pltpu.VMEM((128, DIM), jnp.float32),
        pltpu.SemaphoreType.DMA,
    ],
)
def _s4_gather_y(pos_hbm, yp_hbm, y_hbm, pos_v, rows_v, sem):
    w = _wid()
    tbase = w * CHUNK
    pltpu.sync_copy(pos_hbm.at[w], pos_v)

    def body(j, carry):
        pltpu.async_copy(yp_hbm.at[pos_v.at[j]], rows_v, sem).wait()
        pltpu.sync_copy(rows_v, y_hbm.at[pl.ds(tbase + j * 128, 128)])
        return carry

    lax.fori_loop(0, CHUNK // 128, body, 0)


def _tc_body(wid_ref, valid_ref, x_ref, w0_ref, w1_ref, g_ref, y_ref):
    i = pl.program_id(0)

    @pl.when(valid_ref[i] != 0)
    def _():
        u = jnp.dot(x_ref[...], w0_ref[0], preferred_element_type=jnp.float32)
        h = 0.5 * u * (1.0 + lax.erf(u * 0.7071067811865476))
        o = jnp.dot(h, w1_ref[0], preferred_element_type=jnp.float32)
        y_ref[...] = o * g_ref[...]


def _tc_call(wid_arr, valid_arr, xp, W0b, W1b, gate_pad):
    grid_spec = pltpu.PrefetchScalarGridSpec(
        num_scalar_prefetch=2,
        grid=(MAX_TILES,),
        in_specs=[
            pl.BlockSpec((BM, DIM), lambda i, wid, val: (i, 0)),
            pl.BlockSpec((1, DIM, HID), lambda i, wid, val: (wid[i], 0, 0)),
            pl.BlockSpec((1, HID, DIM), lambda i, wid, val: (wid[i], 0, 0)),
            pl.BlockSpec((BM, 1), lambda i, wid, val: (i, 0)),
        ],
        out_specs=pl.BlockSpec((BM, DIM), lambda i, wid, val: (i, 0)),
    )
    return pl.pallas_call(
        _tc_body,
        grid_spec=grid_spec,
        out_shape=jax.ShapeDtypeStruct((MPAD, DIM), jnp.float32),
    )(wid_arr, valid_arr, xp, W0b, W1b, gate_pad)


def kernel(x, routing_weights, routing_indices, W0, W1):
    e = routing_indices[:, 0].astype(jnp.int32)

    counts_chunks = _s1_count(e)
    counts = counts_chunks.sum(axis=0)
    tiles_pe = (counts + BM - 1) // BM
    cum_tiles = jnp.cumsum(tiles_pe)
    num_valid = cum_tiles[-1]
    base_slot = (cum_tiles - tiles_pe) * BM
    csum = jnp.cumsum(counts_chunks, axis=0) - counts_chunks
    offs = (base_slot[None, :] + csum).astype(jnp.int32)

    ti = jnp.arange(MAX_TILES, dtype=jnp.int32)
    eot = jnp.searchsorted(cum_tiles, ti, side="right").astype(jnp.int32)
    wid_arr = jnp.minimum(eot, E - 1)
    valid_arr = (ti < num_valid).astype(jnp.int32)

    perm, gate_pad, pos = _s2_place(e, routing_weights.reshape(-1), offs)
    xp = _s3_gather_x(perm.reshape(NW, SROWS, 128), x)
    yp = _tc_call(wid_arr, valid_arr, xp, W0, W1, gate_pad.reshape(MPAD, 1))
    return _s4_gather_y(pos, yp)


# speedup vs baseline: 1.7328x; 1.7328x over previous
import functools

import jax
import jax.numpy as jnp
from jax import lax
from jax.experimental import pallas as pl
from jax.experimental.pallas import tpu as pltpu
from jax.experimental.pallas import tpu_sc as plsc

DIM = 768
HID = 1536
E = 64
N = 32768

NC = 2
NS = 16
NW = NC * NS
CHUNK = N // NW

BM = 256
MAX_TILES = N // BM + E
MPAD = MAX_TILES * BM
SLOTS_H = MPAD // NW
GR = 64
S3_PAIRS = SLOTS_H // (2 * GR)

_mesh = plsc.VectorSubcoreMesh(core_axis_name="c", subcore_axis_name="s")


def _wid():
    return lax.axis_index("s") * NC + lax.axis_index("c")


SC_BASE = 1


@functools.partial(
    pl.kernel,
    mesh=_mesh,
    out_type=jax.ShapeDtypeStruct((NW, E), jnp.int32),
    scratch_types=[
        pltpu.VMEM((CHUNK,), jnp.int32),
        pltpu.VMEM((E,), jnp.int32),
    ],
    compiler_params=pltpu.CompilerParams(needs_layout_passes=False),
)
def _s1_count(e_hbm, counts_hbm, e_v, cnt_v):
    w = _wid()
    pltpu.sync_copy(e_hbm.at[pl.ds(w * CHUNK, CHUNK)], e_v)
    zeros = jnp.zeros((16,), jnp.int32)
    for j in range(E // 16):
        cnt_v[pl.ds(j * 16, 16)] = zeros

    def body(k, carry):
        e16 = e_v[pl.ds(k * 16, 16)]
        occ, last = plsc.scan_count(e16)
        old = plsc.load_gather(cnt_v, [e16])
        plsc.store_scatter(cnt_v, [e16], old + occ + (1 - SC_BASE), mask=last)
        return carry

    lax.fori_loop(0, CHUNK // 16, body, 0)
    pltpu.sync_copy(cnt_v, counts_hbm.at[w])


@functools.partial(
    pl.kernel,
    mesh=_mesh,
    out_type=(
        jax.ShapeDtypeStruct((MPAD,), jnp.int32),
        jax.ShapeDtypeStruct((MPAD,), jnp.float32),
        jax.ShapeDtypeStruct((NW, 8, 128), jnp.int32),
    ),
    scratch_types=[
        pltpu.VMEM((CHUNK,), jnp.int32),
        pltpu.VMEM((E,), jnp.int32),
        pltpu.VMEM((8, 128), jnp.int32),
        pltpu.VMEM((8, 128), jnp.int32),
        pltpu.VMEM((8, 128), jnp.int32),
        pltpu.VMEM((8, 128), jnp.float32),
        pltpu.SemaphoreType.DMA,
    ],
    compiler_params=pltpu.CompilerParams(needs_layout_passes=False),
)
def _s2_place(e_hbm, rwflat_hbm, offs_hbm, perm_hbm, gatep_hbm, pos_hbm,
              e_v, cur_v, pos_v, tok_v, gidx_v, gate_v, sem):
    w = _wid()
    base = w * CHUNK
    pltpu.sync_copy(e_hbm.at[pl.ds(base, CHUNK)], e_v)
    pltpu.sync_copy(offs_hbm.at[w], cur_v)

    iota16 = lax.iota(jnp.int32, 16)
    copies = []
    for k in range(CHUNK // 16):
        e16 = e_v[pl.ds(k * 16, 16)]
        occ, last = plsc.scan_count(e16)
        old = plsc.load_gather(cur_v, [e16])
        pos16 = old + occ - SC_BASE
        plsc.store_scatter(cur_v, [e16], old + occ + (1 - SC_BASE), mask=last)
        loc16 = iota16 + (k * 16)
        tok16 = loc16 + base
        r, c = k // 8, (k % 8) * 16
        pos_v[r, pl.ds(c, 16)] = pos16
        tok_v[r, pl.ds(c, 16)] = tok16
        gidx_v[r, pl.ds(c, 16)] = tok16 * E + e16
        if k % 8 == 7:
            copies.append(pltpu.async_copy(
                rwflat_hbm.at[gidx_v.at[r]], gate_v.at[r], sem))
            copies.append(pltpu.async_copy(
                tok_v.at[r], perm_hbm.at[pos_v.at[r]], sem))
    for cp in copies:
        cp.wait()

    copies = [
        pltpu.async_copy(gate_v.at[j], gatep_hbm.at[pos_v.at[j]], sem)
        for j in range(8)
    ]
    for cp in copies:
        cp.wait()
    pltpu.sync_copy(pos_v, pos_hbm.at[w])


SROWS = SLOTS_H // 128


@functools.partial(
    pl.kernel,
    mesh=_mesh,
    out_type=jax.ShapeDtypeStruct((MPAD, DIM), jnp.float32),
    scratch_types=[
        pltpu.VMEM((SROWS, 128), jnp.int32),
        pltpu.VMEM((128, DIM), jnp.float32),
        pltpu.SemaphoreType.DMA,
    ],
)
def _s3_gather_x(perm_hbm, x_hbm, xp_hbm, perm_v, rows_v, sem):
    w = _wid()
    sbase = w * SLOTS_H
    pltpu.sync_copy(perm_hbm.at[w], perm_v)
    iota16 = lax.iota(jnp.int32, 16)
    for r in range(SROWS):
        for k in range(8):
            v = perm_v[r, pl.ds(k * 16, 16)]
            fallback = ((sbase + r * 128 + k * 16) & (N - 1)) + iota16
            ok = (v >= 0) & (v < N)
            perm_v[r, pl.ds(k * 16, 16)] = jnp.where(ok, v, fallback)

    def body(j, carry):
        pltpu.async_copy(x_hbm.at[perm_v.at[j]], rows_v, sem).wait()
        pltpu.sync_copy(rows_v, xp_hbm.at[pl.ds(sbase + j * 128, 128)])
        return carry

    lax.fori_loop(0, SROWS, body, 0)


@functools.partial(
    pl.kernel,
    mesh=_mesh,
    out_type=jax.ShapeDtypeStruct((N, DIM), jnp.float32),
    scratch_types=[
        pltpu.VMEM((8, 128), jnp.int32),
        pltpu.VMEM((128, DIM), jnp.float32),
        pltpu.SemaphoreType.DMA,
    ],
)
def _s4_gather_y(pos_hbm, yp_hbm, y_hbm, pos_v, rows_v, sem):
    w = _wid()
    tbase = w * CHUNK
    pltpu.sync_copy(pos_hbm.at[w], pos_v)

    def body(j, carry):
        pltpu.async_copy(yp_hbm.at[pos_v.at[j]], rows_v, sem).wait()
        pltpu.sync_copy(rows_v, y_hbm.at[pl.ds(tbase + j * 128, 128)])
        return carry

    lax.fori_loop(0, CHUNK // 128, body, 0)


def _tc_body(wid_ref, valid_ref, x_ref, w0_ref, w1_ref, g_ref, y_ref):
    i = pl.program_id(0)

    @pl.when(valid_ref[i] != 0)
    def _():
        u = jnp.dot(x_ref[...], w0_ref[0], preferred_element_type=jnp.float32)
        h = 0.5 * u * (1.0 + lax.erf(u * 0.7071067811865476))
        o = jnp.dot(h, w1_ref[0], preferred_element_type=jnp.float32)
        y_ref[...] = o * g_ref[...]


def _tc_call(wid_arr, valid_arr, xp, W0b, W1b, gate_pad):
    grid_spec = pltpu.PrefetchScalarGridSpec(
        num_scalar_prefetch=2,
        grid=(MAX_TILES,),
        in_specs=[
            pl.BlockSpec((BM, DIM), lambda i, wid, val: (i, 0)),
            pl.BlockSpec((1, DIM, HID), lambda i, wid, val: (wid[i], 0, 0)),
            pl.BlockSpec((1, HID, DIM), lambda i, wid, val: (wid[i], 0, 0)),
            pl.BlockSpec((BM, 1), lambda i, wid, val: (i, 0)),
        ],
        out_specs=pl.BlockSpec((BM, DIM), lambda i, wid, val: (i, 0)),
    )
    return pl.pallas_call(
        _tc_body,
        grid_spec=grid_spec,
        out_shape=jax.ShapeDtypeStruct((MPAD, DIM), jnp.float32),
    )(wid_arr, valid_arr, xp, W0b, W1b, gate_pad)


def kernel(x, routing_weights, routing_indices, W0, W1):
    e = routing_indices[:, 0].astype(jnp.int32)

    counts_chunks = _s1_count(e)
    counts = counts_chunks.sum(axis=0)
    tiles_pe = (counts + BM - 1) // BM
    cum_tiles = jnp.cumsum(tiles_pe)
    num_valid = cum_tiles[-1]
    base_slot = (cum_tiles - tiles_pe) * BM
    csum = jnp.cumsum(counts_chunks, axis=0) - counts_chunks
    offs = (base_slot[None, :] + csum).astype(jnp.int32)

    ti = jnp.arange(MAX_TILES, dtype=jnp.int32)
    eot = jnp.searchsorted(cum_tiles, ti, side="right").astype(jnp.int32)
    wid_arr = jnp.minimum(eot, E - 1)
    valid_arr = (ti < num_valid).astype(jnp.int32)

    perm, gate_pad, pos = _s2_place(e, routing_weights.reshape(-1), offs)
    xp = _s3_gather_x(perm.reshape(NW, SROWS, 128), x)
    yp = _tc_call(wid_arr, valid_arr, xp, W0, W1, gate_pad.reshape(MPAD, 1))
    return _s4_gather_y(pos, yp)
